# Initial kernel scaffold; baseline (speedup 1.0000x reference)
import jax
import jax.numpy as jnp
from jax.experimental import pallas as pl


def kernel(x, edge_index, Wl0, Wr0, b0, gamma0, beta0, Wl1, Wr1, b1, gamma1, beta1, Wl2, Wr2, b2, gamma2, beta2):
    raise NotImplementedError("write your pallas kernel here")


# speedup vs baseline: 4.1136x; 4.1136x over previous
import functools

import jax
import jax.numpy as jnp
from jax import lax
from jax.experimental import pallas as pl
from jax.experimental.pallas import tpu as pltpu
from jax.experimental.pallas import tpu_sc as plsc

N = 10000
E = 320000
H = 512

NSUB = 16
NCORE = 2
EB = 80
SB = 25
EPW = E // NSUB
NBLK = EPW // EB
RPW = 624
ZR = 16


def _fill2(ref, val):
    R, C = ref.shape

    def body(r, carry):
        for j in range(C // 16):
            ref[r, pl.ds(j * 16, 16)] = jnp.full((16,), val, ref.dtype)
        return carry

    lax.fori_loop(0, R, body, 0)


def _fill1(ref, val):
    n = ref.shape[0]

    def body(r, carry):
        ref[pl.ds(r * 16, 16)] = jnp.full((16,), val, ref.dtype)
        return carry

    lax.fori_loop(0, n // 16, body, 0)


def _make_agg(F, nch, edge_split):
    nch_per_core = 1 if edge_split else nch // NCORE
    nblk = (E // (NCORE * NSUB)) // EB if edge_split else NBLK
    nsb = nblk // SB
    mesh = plsc.VectorSubcoreMesh(core_axis_name="c", subcore_axis_name="s")

    n_out = 4 if edge_split else nch
    out_type = [jax.ShapeDtypeStruct((N, F), jnp.float32)
                for _ in range(n_out)]

    scratch_types = [
        pltpu.VMEM((SB, EB), jnp.int32),
        pltpu.VMEM((SB, EB), jnp.int32),
        pltpu.VMEM((EB, F), jnp.float32),
        pltpu.VMEM((ZR, F), jnp.float32),
        pltpu.SemaphoreType.DMA,
        pltpu.VMEM_SHARED((N, F), jnp.float32),
    ]

    def body(*refs):
        xcs = refs[:nch]
        src4, dst4 = refs[nch], refs[nch + 1]
        outs = refs[nch + 2:nch + 2 + n_out]
        pos = nch + 2 + n_out
        sidx, didx, gbuf, zbuf, sem, acc = refs[pos:pos + 6]

        c = lax.axis_index("c")
        sid = lax.axis_index("s")

        _fill2(zbuf, 0.0)
        widx = c * NSUB + sid if edge_split else sid

        def zero_acc():
            nz = jnp.where(sid == NSUB - 1, (N - (NSUB - 1) * RPW) // ZR,
                           RPW // ZR)

            def zrow(k, carry):
                pltpu.sync_copy(zbuf, acc.at[pl.ds(sid * RPW + k * ZR, ZR)])
                return carry

            lax.fori_loop(0, nz, zrow, 0)

        def write_out(sc_hbm):
            @pl.when(sid < NSUB - 1)
            def _():
                pltpu.sync_copy(acc.at[pl.ds(sid * RPW, RPW)],
                                sc_hbm.at[pl.ds(sid * RPW, RPW)])

            @pl.when(sid == NSUB - 1)
            def _():
                tail = N - (NSUB - 1) * RPW
                pltpu.sync_copy(acc.at[pl.ds((NSUB - 1) * RPW, tail)],
                                sc_hbm.at[pl.ds((NSUB - 1) * RPW, tail)])

        def run_chunk(xc_hbm, sc_hbm):
            zero_acc()
            plsc.subcore_barrier()

            def sbk(j, carry):
                pltpu.sync_copy(src4.at[widx, j], sidx)
                pltpu.sync_copy(dst4.at[widx, j], didx)

                def blk(b, carry2):
                    pltpu.async_copy(xc_hbm.at[sidx.at[b]], gbuf, sem).wait()
                    pltpu.sync_copy(gbuf, acc.at[didx.at[b]], add=True)
                    return carry2

                lax.fori_loop(0, SB, blk, 0)
                return carry

            lax.fori_loop(0, nsb, sbk, 0)
            plsc.subcore_barrier()
            write_out(sc_hbm)
            plsc.subcore_barrier()

        def run_deg(dc_hbm):
            zero_acc()
            _fill2(gbuf, 1.0)
            plsc.subcore_barrier()

            def sbk(j, carry):
                pltpu.sync_copy(dst4.at[widx, j], didx)

                def blk(b, carry2):
                    pltpu.sync_copy(gbuf, acc.at[didx.at[b]], add=True)
                    return carry2

                lax.fori_loop(0, SB, blk, 0)
                return carry

            lax.fori_loop(0, nsb, sbk, 0)
            plsc.subcore_barrier()
            write_out(dc_hbm)
            plsc.subcore_barrier()

        for core in range(NCORE):
            @pl.when(c == core)
            def _(core=core):
                if edge_split:
                    run_chunk(xcs[0], outs[core])
                    run_deg(outs[NCORE + core])
                else:
                    for j in range(nch_per_core):
                        k = core * nch_per_core + j
                        run_chunk(xcs[k], outs[k])

    return pl.kernel(body, out_type=out_type, mesh=mesh,
                     scratch_types=scratch_types)


def _dense_tc(din, F, nch, parts, bN=1000):
    grid = N // bN

    def body(*refs):
        pos = 0
        deg_refs = refs[pos:pos + parts]; pos += parts
        s_refs = refs[pos:pos + nch * parts]; pos += nch * parts
        x_refs = refs[pos:pos + nch]; pos += nch
        Wl_ref, Wr_ref, b_ref = refs[pos:pos + 3]; pos += 3
        h_ref, ssum_ref, ssq_ref = refs[pos:pos + 3]; pos += 3

        if parts == 2:
            d = deg_refs[0][:, :1] + deg_refs[1][:, :1]
            refs[pos][...] = d
        else:
            d = deg_refs[0][...]
        invd = 1.0 / jnp.clip(d, 1.0, None)
        h = jnp.broadcast_to(b_ref[...], (bN, H))
        for j in range(nch):
            s = s_refs[j * parts][...]
            for p in range(1, parts):
                s = s + s_refs[j * parts + p][...]
            h = h + jnp.dot(s * invd, Wl_ref[pl.ds(j * F, F), :],
                            preferred_element_type=jnp.float32)
            h = h + jnp.dot(x_refs[j][...], Wr_ref[pl.ds(j * F, F), :],
                            preferred_element_type=jnp.float32)
        h_ref[...] = h

        @pl.when(pl.program_id(0) == 0)
        def _():
            ssum_ref[...] = jnp.zeros((1, H), jnp.float32)
            ssq_ref[...] = jnp.zeros((1, H), jnp.float32)

        ssum_ref[...] += jnp.sum(h, axis=0, keepdims=True)
        ssq_ref[...] += jnp.sum(h * h, axis=0, keepdims=True)

    degw = F if parts == 2 else 1
    in_specs = (
        [pl.BlockSpec((bN, degw), lambda i: (i, 0))] * parts
        + [pl.BlockSpec((bN, F), lambda i: (i, 0))
           for _ in range(nch * parts + nch)]
        + [pl.BlockSpec((din, H), lambda i: (0, 0))] * 2
        + [pl.BlockSpec((1, H), lambda i: (0, 0))]
    )
    out_specs = [
        pl.BlockSpec((bN, H), lambda i: (i, 0)),
        pl.BlockSpec((1, H), lambda i: (0, 0)),
        pl.BlockSpec((1, H), lambda i: (0, 0)),
    ]
    out_shape = [
        jax.ShapeDtypeStruct((N, H), jnp.float32),
        jax.ShapeDtypeStruct((1, H), jnp.float32),
        jax.ShapeDtypeStruct((1, H), jnp.float32),
    ]
    if parts == 2:
        out_specs.append(pl.BlockSpec((bN, 1), lambda i: (i, 0)))
        out_shape.append(jax.ShapeDtypeStruct((N, 1), jnp.float32))
    return pl.pallas_call(body, grid=(grid,), in_specs=in_specs,
                          out_specs=out_specs, out_shape=out_shape)


def _bn_elu_tc(Fout, nout, bN=1000):
    grid = N // bN

    def body(h_ref, ssum_ref, ssq_ref, g_ref, bt_ref, *out_refs):
        mu = ssum_ref[...] * (1.0 / N)
        var = ssq_ref[...] * (1.0 / N) - mu * mu
        scale = g_ref[...] * lax.rsqrt(var + 1e-5)
        shift = bt_ref[...] - mu * scale
        t = h_ref[...] * scale + shift
        o = jnp.where(t > 0, t, jnp.exp(jnp.minimum(t, 0.0)) - 1.0)
        for j in range(nout):
            out_refs[j][...] = o[:, j * Fout:(j + 1) * Fout]

    in_specs = [
        pl.BlockSpec((bN, H), lambda i: (i, 0)),
        pl.BlockSpec((1, H), lambda i: (0, 0)),
        pl.BlockSpec((1, H), lambda i: (0, 0)),
        pl.BlockSpec((1, H), lambda i: (0, 0)),
        pl.BlockSpec((1, H), lambda i: (0, 0)),
    ]
    out_specs = [pl.BlockSpec((bN, Fout), lambda i: (i, 0))] * nout
    out_shape = [jax.ShapeDtypeStruct((N, Fout), jnp.float32)] * nout
    return pl.pallas_call(body, grid=(grid,), in_specs=in_specs,
                          out_specs=out_specs, out_shape=out_shape)


def kernel(x, edge_index, Wl0, Wr0, b0, gamma0, beta0,
           Wl1, Wr1, b1, gamma1, beta1, Wl2, Wr2, b2, gamma2, beta2):
    src16 = edge_index[0].reshape(NSUB, NBLK // SB, SB, EB)
    dst16 = edge_index[1].reshape(NSUB, NBLK // SB, SB, EB)
    nblk2 = E // (NCORE * NSUB) // EB
    src32 = edge_index[0].reshape(NCORE * NSUB, nblk2 // SB, SB, EB)
    dst32 = edge_index[1].reshape(NCORE * NSUB, nblk2 // SB, SB, EB)

    s0a, s0b, d0a, d0b = _make_agg(128, 1, True)(x, src32, dst32)
    h, ssum, ssq, deg = _dense_tc(128, 128, 1, 2)(
        d0a, d0b, s0a, s0b, x, Wl0, Wr0, b0.reshape(1, H))
    xcs = _bn_elu_tc(128, 4)(h, ssum, ssq,
                             gamma0.reshape(1, H), beta0.reshape(1, H))

    for Wl, Wr, b, g, bt, last in (
            (Wl1, Wr1, b1, gamma1, beta1, False),
            (Wl2, Wr2, b2, gamma2, beta2, True)):
        scs = _make_agg(128, 4, False)(*xcs, src16, dst16)
        h, ssum, ssq = _dense_tc(512, 128, 4, 1)(
            deg, *scs, *xcs, Wl, Wr, b.reshape(1, H))
        if last:
            (out,) = _bn_elu_tc(H, 1)(h, ssum, ssq,
                                      g.reshape(1, H), bt.reshape(1, H))
        else:
            xcs = _bn_elu_tc(128, 4)(h, ssum, ssq,
                                     g.reshape(1, H), bt.reshape(1, H))
    return out


# speedup vs baseline: 5.1917x; 1.2621x over previous
import functools

import jax
import jax.numpy as jnp
from jax import lax
from jax.experimental import pallas as pl
from jax.experimental.pallas import tpu as pltpu
from jax.experimental.pallas import tpu_sc as plsc

N = 10000
E = 320000
H = 512

NSUB = 16
NCORE = 2
EB = 80
SB = 25
EPW = E // NSUB
NBLK = EPW // EB
RPW = 624
ZR = 16


def _fill2(ref, val):
    R, C = ref.shape

    def body(r, carry):
        for j in range(C // 16):
            ref[r, pl.ds(j * 16, 16)] = jnp.full((16,), val, ref.dtype)
        return carry

    lax.fori_loop(0, R, body, 0)


def _fill1(ref, val):
    n = ref.shape[0]

    def body(r, carry):
        ref[pl.ds(r * 16, 16)] = jnp.full((16,), val, ref.dtype)
        return carry

    lax.fori_loop(0, n // 16, body, 0)


def _make_agg(F, nch, edge_split):
    nch_per_core = 1 if edge_split else nch // NCORE
    nblk = (E // (NCORE * NSUB)) // EB if edge_split else NBLK
    nsb = nblk // SB
    mesh = plsc.VectorSubcoreMesh(core_axis_name="c", subcore_axis_name="s")

    n_out = 4 if edge_split else nch
    out_type = [jax.ShapeDtypeStruct((N, F), jnp.float32)
                for _ in range(n_out)]

    scratch_types = [
        pltpu.VMEM((SB, EB), jnp.int32),
        pltpu.VMEM((SB, EB), jnp.int32),
        pltpu.VMEM((EB, F), jnp.float32),
        pltpu.VMEM((EB, F), jnp.float32),
        pltpu.VMEM((ZR, F), jnp.float32),
        pltpu.SemaphoreType.DMA,
        pltpu.SemaphoreType.DMA,
        pltpu.VMEM_SHARED((N, F), jnp.float32),
    ]

    def body(*refs):
        xcs = refs[:nch]
        src4, dst4 = refs[nch], refs[nch + 1]
        outs = refs[nch + 2:nch + 2 + n_out]
        pos = nch + 2 + n_out
        sidx, didx, gbuf0, gbuf1, zbuf, sem0, sem1, acc = refs[pos:pos + 8]
        gbufs = (gbuf0, gbuf1)
        sems = (sem0, sem1)

        c = lax.axis_index("c")
        sid = lax.axis_index("s")

        _fill2(zbuf, 0.0)
        widx = c * NSUB + sid if edge_split else sid

        def zero_acc():
            nz = jnp.where(sid == NSUB - 1, (N - (NSUB - 1) * RPW) // ZR,
                           RPW // ZR)

            def zrow(k, carry):
                pltpu.sync_copy(zbuf, acc.at[pl.ds(sid * RPW + k * ZR, ZR)])
                return carry

            lax.fori_loop(0, nz, zrow, 0)

        def write_out(sc_hbm):
            @pl.when(sid < NSUB - 1)
            def _():
                pltpu.sync_copy(acc.at[pl.ds(sid * RPW, RPW)],
                                sc_hbm.at[pl.ds(sid * RPW, RPW)])

            @pl.when(sid == NSUB - 1)
            def _():
                tail = N - (NSUB - 1) * RPW
                pltpu.sync_copy(acc.at[pl.ds((NSUB - 1) * RPW, tail)],
                                sc_hbm.at[pl.ds((NSUB - 1) * RPW, tail)])

        def run_chunk(xc_hbm, sc_hbm):
            zero_acc()
            plsc.subcore_barrier()

            def sbk(j, carry):
                pltpu.sync_copy(src4.at[widx, j], sidx)
                pltpu.sync_copy(dst4.at[widx, j], didx)
                cps = [None, None]
                cps[0] = pltpu.async_copy(xc_hbm.at[sidx.at[0]], gbufs[0],
                                          sems[0])
                for b in range(SB):
                    cur = b % 2
                    cps[cur].wait()
                    if b + 1 < SB:
                        cps[1 - cur] = pltpu.async_copy(
                            xc_hbm.at[sidx.at[b + 1]], gbufs[1 - cur],
                            sems[1 - cur])
                    pltpu.sync_copy(gbufs[cur], acc.at[didx.at[b]], add=True)
                return carry

            lax.fori_loop(0, nsb, sbk, 0)
            plsc.subcore_barrier()
            write_out(sc_hbm)
            plsc.subcore_barrier()

        def run_deg(dc_hbm):
            zero_acc()
            _fill2(gbuf0, 1.0)
            plsc.subcore_barrier()

            def sbk(j, carry):
                pltpu.sync_copy(dst4.at[widx, j], didx)

                def blk(b, carry2):
                    pltpu.sync_copy(gbuf0, acc.at[didx.at[b]], add=True)
                    return carry2

                lax.fori_loop(0, SB, blk, 0)
                return carry

            lax.fori_loop(0, nsb, sbk, 0)
            plsc.subcore_barrier()
            write_out(dc_hbm)
            plsc.subcore_barrier()

        for core in range(NCORE):
            @pl.when(c == core)
            def _(core=core):
                if edge_split:
                    run_chunk(xcs[0], outs[core])
                    run_deg(outs[NCORE + core])
                else:
                    for j in range(nch_per_core):
                        k = core * nch_per_core + j
                        run_chunk(xcs[k], outs[k])

    return pl.kernel(body, out_type=out_type, mesh=mesh,
                     scratch_types=scratch_types)


def _dense_tc(din, F, nch, parts, bN=1000):
    grid = N // bN

    def body(*refs):
        pos = 0
        deg_refs = refs[pos:pos + parts]; pos += parts
        s_refs = refs[pos:pos + nch * parts]; pos += nch * parts
        x_refs = refs[pos:pos + nch]; pos += nch
        Wl_ref, Wr_ref, b_ref = refs[pos:pos + 3]; pos += 3
        h_ref, ssum_ref, ssq_ref = refs[pos:pos + 3]; pos += 3

        if parts == 2:
            d = deg_refs[0][:, :1] + deg_refs[1][:, :1]
            refs[pos][...] = d
        else:
            d = deg_refs[0][...]
        invd = 1.0 / jnp.clip(d, 1.0, None)
        h = jnp.broadcast_to(b_ref[...], (bN, H))
        for j in range(nch):
            s = s_refs[j * parts][...]
            for p in range(1, parts):
                s = s + s_refs[j * parts + p][...]
            h = h + jnp.dot(s * invd, Wl_ref[pl.ds(j * F, F), :],
                            preferred_element_type=jnp.float32)
            h = h + jnp.dot(x_refs[j][...], Wr_ref[pl.ds(j * F, F), :],
                            preferred_element_type=jnp.float32)
        h_ref[...] = h

        @pl.when(pl.program_id(0) == 0)
        def _():
            ssum_ref[...] = jnp.zeros((1, H), jnp.float32)
            ssq_ref[...] = jnp.zeros((1, H), jnp.float32)

        ssum_ref[...] += jnp.sum(h, axis=0, keepdims=True)
        ssq_ref[...] += jnp.sum(h * h, axis=0, keepdims=True)

    degw = F if parts == 2 else 1
    in_specs = (
        [pl.BlockSpec((bN, degw), lambda i: (i, 0))] * parts
        + [pl.BlockSpec((bN, F), lambda i: (i, 0))
           for _ in range(nch * parts + nch)]
        + [pl.BlockSpec((din, H), lambda i: (0, 0))] * 2
        + [pl.BlockSpec((1, H), lambda i: (0, 0))]
    )
    out_specs = [
        pl.BlockSpec((bN, H), lambda i: (i, 0)),
        pl.BlockSpec((1, H), lambda i: (0, 0)),
        pl.BlockSpec((1, H), lambda i: (0, 0)),
    ]
    out_shape = [
        jax.ShapeDtypeStruct((N, H), jnp.float32),
        jax.ShapeDtypeStruct((1, H), jnp.float32),
        jax.ShapeDtypeStruct((1, H), jnp.float32),
    ]
    if parts == 2:
        out_specs.append(pl.BlockSpec((bN, 1), lambda i: (i, 0)))
        out_shape.append(jax.ShapeDtypeStruct((N, 1), jnp.float32))
    return pl.pallas_call(body, grid=(grid,), in_specs=in_specs,
                          out_specs=out_specs, out_shape=out_shape)


def _bn_elu_tc(Fout, nout, bN=1000):
    grid = N // bN

    def body(h_ref, ssum_ref, ssq_ref, g_ref, bt_ref, *out_refs):
        mu = ssum_ref[...] * (1.0 / N)
        var = ssq_ref[...] * (1.0 / N) - mu * mu
        scale = g_ref[...] * lax.rsqrt(var + 1e-5)
        shift = bt_ref[...] - mu * scale
        t = h_ref[...] * scale + shift
        o = jnp.where(t > 0, t, jnp.exp(jnp.minimum(t, 0.0)) - 1.0)
        for j in range(nout):
            out_refs[j][...] = o[:, j * Fout:(j + 1) * Fout]

    in_specs = [
        pl.BlockSpec((bN, H), lambda i: (i, 0)),
        pl.BlockSpec((1, H), lambda i: (0, 0)),
        pl.BlockSpec((1, H), lambda i: (0, 0)),
        pl.BlockSpec((1, H), lambda i: (0, 0)),
        pl.BlockSpec((1, H), lambda i: (0, 0)),
    ]
    out_specs = [pl.BlockSpec((bN, Fout), lambda i: (i, 0))] * nout
    out_shape = [jax.ShapeDtypeStruct((N, Fout), jnp.float32)] * nout
    return pl.pallas_call(body, grid=(grid,), in_specs=in_specs,
                          out_specs=out_specs, out_shape=out_shape)


def kernel(x, edge_index, Wl0, Wr0, b0, gamma0, beta0,
           Wl1, Wr1, b1, gamma1, beta1, Wl2, Wr2, b2, gamma2, beta2):
    src16 = edge_index[0].reshape(NSUB, NBLK // SB, SB, EB)
    dst16 = edge_index[1].reshape(NSUB, NBLK // SB, SB, EB)
    nblk2 = E // (NCORE * NSUB) // EB
    src32 = edge_index[0].reshape(NCORE * NSUB, nblk2 // SB, SB, EB)
    dst32 = edge_index[1].reshape(NCORE * NSUB, nblk2 // SB, SB, EB)

    s0a, s0b, d0a, d0b = _make_agg(128, 1, True)(x, src32, dst32)
    h, ssum, ssq, deg = _dense_tc(128, 128, 1, 2)(
        d0a, d0b, s0a, s0b, x, Wl0, Wr0, b0.reshape(1, H))
    xcs = _bn_elu_tc(128, 4)(h, ssum, ssq,
                             gamma0.reshape(1, H), beta0.reshape(1, H))

    for Wl, Wr, b, g, bt, last in (
            (Wl1, Wr1, b1, gamma1, beta1, False),
            (Wl2, Wr2, b2, gamma2, beta2, True)):
        scs = _make_agg(128, 4, False)(*xcs, src16, dst16)
        h, ssum, ssq = _dense_tc(512, 128, 4, 1)(
            deg, *scs, *xcs, Wl, Wr, b.reshape(1, H))
        if last:
            (out,) = _bn_elu_tc(H, 1)(h, ssum, ssq,
                                      g.reshape(1, H), bt.reshape(1, H))
        else:
            xcs = _bn_elu_tc(128, 4)(h, ssum, ssq,
                                     g.reshape(1, H), bt.reshape(1, H))
    return out


# speedup vs baseline: 6.0836x; 1.1718x over previous
import functools

import jax
import jax.numpy as jnp
from jax import lax
from jax.experimental import pallas as pl
from jax.experimental.pallas import tpu as pltpu
from jax.experimental.pallas import tpu_sc as plsc

N = 10000
E = 320000
H = 512

NSUB = 16
NCORE = 2
EB = 125
SB = 16
EPW = E // NSUB
NBLK = EPW // EB
RPW = 624
ZR = 16


def _fill2(ref, val):
    R, C = ref.shape

    def body(r, carry):
        for j in range(C // 16):
            ref[r, pl.ds(j * 16, 16)] = jnp.full((16,), val, ref.dtype)
        return carry

    lax.fori_loop(0, R, body, 0)


def _fill1(ref, val):
    n = ref.shape[0]

    def body(r, carry):
        ref[pl.ds(r * 16, 16)] = jnp.full((16,), val, ref.dtype)
        return carry

    lax.fori_loop(0, n // 16, body, 0)


def _make_agg(F, nch, edge_split):
    nch_per_core = 1 if edge_split else nch // NCORE
    nblk = (E // (NCORE * NSUB)) // EB if edge_split else NBLK
    nsb = nblk // SB
    mesh = plsc.VectorSubcoreMesh(core_axis_name="c", subcore_axis_name="s")

    n_out = 4 if edge_split else nch
    out_type = [jax.ShapeDtypeStruct((N, F), jnp.float32)
                for _ in range(n_out)]

    scratch_types = [
        pltpu.VMEM((SB, EB), jnp.int32),
        pltpu.VMEM((SB, EB), jnp.int32),
        pltpu.VMEM((EB, F), jnp.float32),
        pltpu.VMEM((EB, F), jnp.float32),
        pltpu.VMEM((ZR, F), jnp.float32),
        pltpu.SemaphoreType.DMA,
        pltpu.SemaphoreType.DMA,
        pltpu.VMEM_SHARED((N, F), jnp.float32),
    ]

    def body(*refs):
        xcs = refs[:nch]
        src4, dst4 = refs[nch], refs[nch + 1]
        outs = refs[nch + 2:nch + 2 + n_out]
        pos = nch + 2 + n_out
        sidx, didx, gbuf0, gbuf1, zbuf, sem0, sem1, acc = refs[pos:pos + 8]
        gbufs = (gbuf0, gbuf1)
        sems = (sem0, sem1)

        c = lax.axis_index("c")
        sid = lax.axis_index("s")

        _fill2(zbuf, 0.0)
        widx = c * NSUB + sid if edge_split else sid

        def zero_acc():
            nz = jnp.where(sid == NSUB - 1, (N - (NSUB - 1) * RPW) // ZR,
                           RPW // ZR)

            def zrow(k, carry):
                pltpu.sync_copy(zbuf, acc.at[pl.ds(sid * RPW + k * ZR, ZR)])
                return carry

            lax.fori_loop(0, nz, zrow, 0)

        def write_out(sc_hbm):
            @pl.when(sid < NSUB - 1)
            def _():
                pltpu.sync_copy(acc.at[pl.ds(sid * RPW, RPW)],
                                sc_hbm.at[pl.ds(sid * RPW, RPW)])

            @pl.when(sid == NSUB - 1)
            def _():
                tail = N - (NSUB - 1) * RPW
                pltpu.sync_copy(acc.at[pl.ds((NSUB - 1) * RPW, tail)],
                                sc_hbm.at[pl.ds((NSUB - 1) * RPW, tail)])

        def run_chunk(xc_hbm, sc_hbm):
            zero_acc()
            plsc.subcore_barrier()

            def sbk(j, carry):
                pltpu.sync_copy(src4.at[widx, j], sidx)
                pltpu.sync_copy(dst4.at[widx, j], didx)
                cps = [None, None]
                cps[0] = pltpu.async_copy(xc_hbm.at[sidx.at[0]], gbufs[0],
                                          sems[0])
                for b in range(SB):
                    cur = b % 2
                    cps[cur].wait()
                    if b + 1 < SB:
                        cps[1 - cur] = pltpu.async_copy(
                            xc_hbm.at[sidx.at[b + 1]], gbufs[1 - cur],
                            sems[1 - cur])
                    pltpu.sync_copy(gbufs[cur], acc.at[didx.at[b]], add=True)
                return carry

            lax.fori_loop(0, nsb, sbk, 0)
            plsc.subcore_barrier()
            write_out(sc_hbm)
            plsc.subcore_barrier()

        def run_deg(dc_hbm):
            zero_acc()
            _fill2(gbuf0, 1.0)
            plsc.subcore_barrier()

            def sbk(j, carry):
                pltpu.sync_copy(dst4.at[widx, j], didx)

                def blk(b, carry2):
                    pltpu.sync_copy(gbuf0, acc.at[didx.at[b]], add=True)
                    return carry2

                lax.fori_loop(0, SB, blk, 0)
                return carry

            lax.fori_loop(0, nsb, sbk, 0)
            plsc.subcore_barrier()
            write_out(dc_hbm)
            plsc.subcore_barrier()

        for core in range(NCORE):
            @pl.when(c == core)
            def _(core=core):
                if edge_split:
                    run_chunk(xcs[0], outs[core])
                    run_deg(outs[NCORE + core])
                else:
                    for j in range(nch_per_core):
                        k = core * nch_per_core + j
                        run_chunk(xcs[k], outs[k])

    return pl.kernel(body, out_type=out_type, mesh=mesh,
                     scratch_types=scratch_types)


def _dense_tc(din, F, nch, parts, bN=1000):
    grid = N // bN

    def body(*refs):
        pos = 0
        deg_refs = refs[pos:pos + parts]; pos += parts
        s_refs = refs[pos:pos + nch * parts]; pos += nch * parts
        x_refs = refs[pos:pos + nch]; pos += nch
        Wl_ref, Wr_ref, b_ref = refs[pos:pos + 3]; pos += 3
        h_ref, ssum_ref, ssq_ref = refs[pos:pos + 3]; pos += 3

        if parts == 2:
            d = deg_refs[0][:, :1] + deg_refs[1][:, :1]
            refs[pos][...] = d
        else:
            d = deg_refs[0][...]
        invd = 1.0 / jnp.clip(d, 1.0, None)
        h = jnp.broadcast_to(b_ref[...], (bN, H))
        for j in range(nch):
            s = s_refs[j * parts][...]
            for p in range(1, parts):
                s = s + s_refs[j * parts + p][...]
            h = h + jnp.dot(s * invd, Wl_ref[pl.ds(j * F, F), :],
                            preferred_element_type=jnp.float32)
            h = h + jnp.dot(x_refs[j][...], Wr_ref[pl.ds(j * F, F), :],
                            preferred_element_type=jnp.float32)
        h_ref[...] = h

        @pl.when(pl.program_id(0) == 0)
        def _():
            ssum_ref[...] = jnp.zeros((1, H), jnp.float32)
            ssq_ref[...] = jnp.zeros((1, H), jnp.float32)

        ssum_ref[...] += jnp.sum(h, axis=0, keepdims=True)
        ssq_ref[...] += jnp.sum(h * h, axis=0, keepdims=True)

    degw = F if parts == 2 else 1
    in_specs = (
        [pl.BlockSpec((bN, degw), lambda i: (i, 0))] * parts
        + [pl.BlockSpec((bN, F), lambda i: (i, 0))
           for _ in range(nch * parts + nch)]
        + [pl.BlockSpec((din, H), lambda i: (0, 0))] * 2
        + [pl.BlockSpec((1, H), lambda i: (0, 0))]
    )
    out_specs = [
        pl.BlockSpec((bN, H), lambda i: (i, 0)),
        pl.BlockSpec((1, H), lambda i: (0, 0)),
        pl.BlockSpec((1, H), lambda i: (0, 0)),
    ]
    out_shape = [
        jax.ShapeDtypeStruct((N, H), jnp.float32),
        jax.ShapeDtypeStruct((1, H), jnp.float32),
        jax.ShapeDtypeStruct((1, H), jnp.float32),
    ]
    if parts == 2:
        out_specs.append(pl.BlockSpec((bN, 1), lambda i: (i, 0)))
        out_shape.append(jax.ShapeDtypeStruct((N, 1), jnp.float32))
    return pl.pallas_call(body, grid=(grid,), in_specs=in_specs,
                          out_specs=out_specs, out_shape=out_shape)


def _bn_elu_tc(Fout, nout, bN=1000):
    grid = N // bN

    def body(h_ref, ssum_ref, ssq_ref, g_ref, bt_ref, *out_refs):
        mu = ssum_ref[...] * (1.0 / N)
        var = ssq_ref[...] * (1.0 / N) - mu * mu
        scale = g_ref[...] * lax.rsqrt(var + 1e-5)
        shift = bt_ref[...] - mu * scale
        t = h_ref[...] * scale + shift
        o = jnp.where(t > 0, t, jnp.exp(jnp.minimum(t, 0.0)) - 1.0)
        for j in range(nout):
            out_refs[j][...] = o[:, j * Fout:(j + 1) * Fout]

    in_specs = [
        pl.BlockSpec((bN, H), lambda i: (i, 0)),
        pl.BlockSpec((1, H), lambda i: (0, 0)),
        pl.BlockSpec((1, H), lambda i: (0, 0)),
        pl.BlockSpec((1, H), lambda i: (0, 0)),
        pl.BlockSpec((1, H), lambda i: (0, 0)),
    ]
    out_specs = [pl.BlockSpec((bN, Fout), lambda i: (i, 0))] * nout
    out_shape = [jax.ShapeDtypeStruct((N, Fout), jnp.float32)] * nout
    return pl.pallas_call(body, grid=(grid,), in_specs=in_specs,
                          out_specs=out_specs, out_shape=out_shape)


def kernel(x, edge_index, Wl0, Wr0, b0, gamma0, beta0,
           Wl1, Wr1, b1, gamma1, beta1, Wl2, Wr2, b2, gamma2, beta2):
    src16 = edge_index[0].reshape(NSUB, NBLK // SB, SB, EB)
    dst16 = edge_index[1].reshape(NSUB, NBLK // SB, SB, EB)
    nblk2 = E // (NCORE * NSUB) // EB
    src32 = edge_index[0].reshape(NCORE * NSUB, nblk2 // SB, SB, EB)
    dst32 = edge_index[1].reshape(NCORE * NSUB, nblk2 // SB, SB, EB)

    s0a, s0b, d0a, d0b = _make_agg(128, 1, True)(x, src32, dst32)
    h, ssum, ssq, deg = _dense_tc(128, 128, 1, 2)(
        d0a, d0b, s0a, s0b, x, Wl0, Wr0, b0.reshape(1, H))
    xcs = _bn_elu_tc(128, 4)(h, ssum, ssq,
                             gamma0.reshape(1, H), beta0.reshape(1, H))

    for Wl, Wr, b, g, bt, last in (
            (Wl1, Wr1, b1, gamma1, beta1, False),
            (Wl2, Wr2, b2, gamma2, beta2, True)):
        scs = _make_agg(128, 4, False)(*xcs, src16, dst16)
        h, ssum, ssq = _dense_tc(512, 128, 4, 1)(
            deg, *scs, *xcs, Wl, Wr, b.reshape(1, H))
        if last:
            (out,) = _bn_elu_tc(H, 1)(h, ssum, ssq,
                                      g.reshape(1, H), bt.reshape(1, H))
        else:
            xcs = _bn_elu_tc(128, 4)(h, ssum, ssq,
                                     g.reshape(1, H), bt.reshape(1, H))
    return out


# speedup vs baseline: 6.0961x; 1.0021x over previous
import functools

import jax
import jax.numpy as jnp
from jax import lax
from jax.experimental import pallas as pl
from jax.experimental.pallas import tpu as pltpu
from jax.experimental.pallas import tpu_sc as plsc

N = 10000
E = 320000
H = 512

NSUB = 16
NCORE = 2
EB = 125
SB = 16
EPW = E // NSUB
NBLK = EPW // EB
RPW = 624
ZR = 16


def _fill2(ref, val):
    R, C = ref.shape

    def body(r, carry):
        for j in range(C // 16):
            ref[r, pl.ds(j * 16, 16)] = jnp.full((16,), val, ref.dtype)
        return carry

    lax.fori_loop(0, R, body, 0)


def _fill1(ref, val):
    n = ref.shape[0]

    def body(r, carry):
        ref[pl.ds(r * 16, 16)] = jnp.full((16,), val, ref.dtype)
        return carry

    lax.fori_loop(0, n // 16, body, 0)


def _make_agg(F, nch, edge_split):
    nch_per_core = 1 if edge_split else nch // NCORE
    nblk = (E // (NCORE * NSUB)) // EB if edge_split else NBLK
    nsb = nblk // SB
    mesh = plsc.VectorSubcoreMesh(core_axis_name="c", subcore_axis_name="s")

    n_out = 4 if edge_split else nch
    out_type = [jax.ShapeDtypeStruct((N, F), jnp.float32)
                for _ in range(n_out)]

    scratch_types = [
        pltpu.VMEM((SB, EB), jnp.int32),
        pltpu.VMEM((SB, EB), jnp.int32),
        pltpu.VMEM((EB, F), jnp.float32),
        pltpu.VMEM((EB, F), jnp.float32),
        pltpu.VMEM((ZR, F), jnp.float32),
        pltpu.SemaphoreType.DMA,
        pltpu.SemaphoreType.DMA,
        pltpu.VMEM_SHARED((N, F), jnp.float32),
    ]

    def body(*refs):
        xcs = refs[:nch]
        src4, dst4 = refs[nch], refs[nch + 1]
        outs = refs[nch + 2:nch + 2 + n_out]
        pos = nch + 2 + n_out
        sidx, didx, gbuf0, gbuf1, zbuf, sem0, sem1, acc = refs[pos:pos + 8]
        gbufs = (gbuf0, gbuf1)
        sems = (sem0, sem1)

        c = lax.axis_index("c")
        sid = lax.axis_index("s")

        _fill2(zbuf, 0.0)
        widx = c * NSUB + sid if edge_split else sid

        def zero_acc():
            nz = jnp.where(sid == NSUB - 1, (N - (NSUB - 1) * RPW) // ZR,
                           RPW // ZR)

            def zrow(k, carry):
                pltpu.sync_copy(zbuf, acc.at[pl.ds(sid * RPW + k * ZR, ZR)])
                return carry

            lax.fori_loop(0, nz, zrow, 0)

        def write_out(sc_hbm):
            @pl.when(sid < NSUB - 1)
            def _():
                pltpu.sync_copy(acc.at[pl.ds(sid * RPW, RPW)],
                                sc_hbm.at[pl.ds(sid * RPW, RPW)])

            @pl.when(sid == NSUB - 1)
            def _():
                tail = N - (NSUB - 1) * RPW
                pltpu.sync_copy(acc.at[pl.ds((NSUB - 1) * RPW, tail)],
                                sc_hbm.at[pl.ds((NSUB - 1) * RPW, tail)])

        def run_chunk(xc_hbm, sc_hbm):
            zero_acc()
            plsc.subcore_barrier()

            def sbk(j, carry):
                pltpu.sync_copy(src4.at[widx, j], sidx)
                pltpu.sync_copy(dst4.at[widx, j], didx)
                cps = [None, None]
                cps[0] = pltpu.async_copy(xc_hbm.at[sidx.at[0]], gbufs[0],
                                          sems[0])
                for b in range(SB):
                    cur = b % 2
                    cps[cur].wait()
                    if b + 1 < SB:
                        cps[1 - cur] = pltpu.async_copy(
                            xc_hbm.at[sidx.at[b + 1]], gbufs[1 - cur],
                            sems[1 - cur])
                    pltpu.sync_copy(gbufs[cur], acc.at[didx.at[b]], add=True)
                return carry

            lax.fori_loop(0, nsb, sbk, 0)
            plsc.subcore_barrier()
            write_out(sc_hbm)
            plsc.subcore_barrier()

        def run_deg(dc_hbm):
            zero_acc()
            _fill2(gbuf0, 1.0)
            plsc.subcore_barrier()

            def sbk(j, carry):
                pltpu.sync_copy(dst4.at[widx, j], didx)

                def blk(b, carry2):
                    pltpu.sync_copy(gbuf0, acc.at[didx.at[b]], add=True)
                    return carry2

                lax.fori_loop(0, SB, blk, 0)
                return carry

            lax.fori_loop(0, nsb, sbk, 0)
            plsc.subcore_barrier()
            write_out(dc_hbm)
            plsc.subcore_barrier()

        for core in range(NCORE):
            @pl.when(c == core)
            def _(core=core):
                if edge_split:
                    run_chunk(xcs[0], outs[core])
                    run_deg(outs[NCORE + core])
                else:
                    for j in range(nch_per_core):
                        k = core * nch_per_core + j
                        run_chunk(xcs[k], outs[k])

    return pl.kernel(body, out_type=out_type, mesh=mesh,
                     scratch_types=scratch_types)


def _xr_tc(din, F, nch, bN=1000):
    grid = N // bN

    def body(*refs):
        x_refs = refs[:nch]
        Wr_ref, b_ref = refs[nch:nch + 2]
        xr_ref = refs[nch + 2]
        h = jnp.broadcast_to(b_ref[...], (bN, H))
        for j in range(nch):
            h = h + jnp.dot(x_refs[j][...], Wr_ref[pl.ds(j * F, F), :],
                            preferred_element_type=jnp.float32)
        xr_ref[...] = h

    in_specs = (
        [pl.BlockSpec((bN, F), lambda i: (i, 0)) for _ in range(nch)]
        + [pl.BlockSpec((din, H), lambda i: (0, 0)),
           pl.BlockSpec((1, H), lambda i: (0, 0))]
    )
    return pl.pallas_call(
        body, grid=(grid,), in_specs=in_specs,
        out_specs=[pl.BlockSpec((bN, H), lambda i: (i, 0))],
        out_shape=[jax.ShapeDtypeStruct((N, H), jnp.float32)])


def _dense_tc(din, F, nch, parts, bN=1000):
    grid = N // bN

    def body(*refs):
        pos = 0
        deg_refs = refs[pos:pos + parts]; pos += parts
        s_refs = refs[pos:pos + nch * parts]; pos += nch * parts
        xr_ref = refs[pos]; pos += 1
        Wl_ref = refs[pos]; pos += 1
        h_ref, ssum_ref, ssq_ref = refs[pos:pos + 3]; pos += 3

        if parts == 2:
            d = deg_refs[0][:, :1] + deg_refs[1][:, :1]
            refs[pos][...] = d
        else:
            d = deg_refs[0][...]
        invd = 1.0 / jnp.clip(d, 1.0, None)
        h = xr_ref[...]
        for j in range(nch):
            s = s_refs[j * parts][...]
            for p in range(1, parts):
                s = s + s_refs[j * parts + p][...]
            h = h + jnp.dot(s * invd, Wl_ref[pl.ds(j * F, F), :],
                            preferred_element_type=jnp.float32)
        h_ref[...] = h

        @pl.when(pl.program_id(0) == 0)
        def _():
            ssum_ref[...] = jnp.zeros((1, H), jnp.float32)
            ssq_ref[...] = jnp.zeros((1, H), jnp.float32)

        ssum_ref[...] += jnp.sum(h, axis=0, keepdims=True)
        ssq_ref[...] += jnp.sum(h * h, axis=0, keepdims=True)

    degw = F if parts == 2 else 1
    in_specs = (
        [pl.BlockSpec((bN, degw), lambda i: (i, 0))] * parts
        + [pl.BlockSpec((bN, F), lambda i: (i, 0))
           for _ in range(nch * parts)]
        + [pl.BlockSpec((bN, H), lambda i: (i, 0)),
           pl.BlockSpec((din, H), lambda i: (0, 0))]
    )
    out_specs = [
        pl.BlockSpec((bN, H), lambda i: (i, 0)),
        pl.BlockSpec((1, H), lambda i: (0, 0)),
        pl.BlockSpec((1, H), lambda i: (0, 0)),
    ]
    out_shape = [
        jax.ShapeDtypeStruct((N, H), jnp.float32),
        jax.ShapeDtypeStruct((1, H), jnp.float32),
        jax.ShapeDtypeStruct((1, H), jnp.float32),
    ]
    if parts == 2:
        out_specs.append(pl.BlockSpec((bN, 1), lambda i: (i, 0)))
        out_shape.append(jax.ShapeDtypeStruct((N, 1), jnp.float32))
    return pl.pallas_call(body, grid=(grid,), in_specs=in_specs,
                          out_specs=out_specs, out_shape=out_shape)


def _bn_elu_tc(Fout, nout, bN=1000):
    grid = N // bN

    def body(h_ref, ssum_ref, ssq_ref, g_ref, bt_ref, *out_refs):
        mu = ssum_ref[...] * (1.0 / N)
        var = ssq_ref[...] * (1.0 / N) - mu * mu
        scale = g_ref[...] * lax.rsqrt(var + 1e-5)
        shift = bt_ref[...] - mu * scale
        t = h_ref[...] * scale + shift
        o = jnp.where(t > 0, t, jnp.exp(jnp.minimum(t, 0.0)) - 1.0)
        for j in range(nout):
            out_refs[j][...] = o[:, j * Fout:(j + 1) * Fout]

    in_specs = [
        pl.BlockSpec((bN, H), lambda i: (i, 0)),
        pl.BlockSpec((1, H), lambda i: (0, 0)),
        pl.BlockSpec((1, H), lambda i: (0, 0)),
        pl.BlockSpec((1, H), lambda i: (0, 0)),
        pl.BlockSpec((1, H), lambda i: (0, 0)),
    ]
    out_specs = [pl.BlockSpec((bN, Fout), lambda i: (i, 0))] * nout
    out_shape = [jax.ShapeDtypeStruct((N, Fout), jnp.float32)] * nout
    return pl.pallas_call(body, grid=(grid,), in_specs=in_specs,
                          out_specs=out_specs, out_shape=out_shape)


def kernel(x, edge_index, Wl0, Wr0, b0, gamma0, beta0,
           Wl1, Wr1, b1, gamma1, beta1, Wl2, Wr2, b2, gamma2, beta2):
    src16 = edge_index[0].reshape(NSUB, NBLK // SB, SB, EB)
    dst16 = edge_index[1].reshape(NSUB, NBLK // SB, SB, EB)
    nblk2 = E // (NCORE * NSUB) // EB
    src32 = edge_index[0].reshape(NCORE * NSUB, nblk2 // SB, SB, EB)
    dst32 = edge_index[1].reshape(NCORE * NSUB, nblk2 // SB, SB, EB)

    s0a, s0b, d0a, d0b = _make_agg(128, 1, True)(x, src32, dst32)
    (xr,) = _xr_tc(128, 128, 1)(x, Wr0, b0.reshape(1, H))
    h, ssum, ssq, deg = _dense_tc(128, 128, 1, 2)(
        d0a, d0b, s0a, s0b, xr, Wl0)
    xcs = _bn_elu_tc(128, 4)(h, ssum, ssq,
                             gamma0.reshape(1, H), beta0.reshape(1, H))

    for Wl, Wr, b, g, bt, last in (
            (Wl1, Wr1, b1, gamma1, beta1, False),
            (Wl2, Wr2, b2, gamma2, beta2, True)):
        scs = _make_agg(128, 4, False)(*xcs, src16, dst16)
        (xr,) = _xr_tc(512, 128, 4)(*xcs, Wr, b.reshape(1, H))
        h, ssum, ssq = _dense_tc(512, 128, 4, 1)(deg, *scs, xr, Wl)
        if last:
            (out,) = _bn_elu_tc(H, 1)(h, ssum, ssq,
                                      g.reshape(1, H), bt.reshape(1, H))
        else:
            xcs = _bn_elu_tc(128, 4)(h, ssum, ssq,
                                     g.reshape(1, H), bt.reshape(1, H))
    return out
